# baseline (speedup 1.0000x reference)
import jax
import jax.numpy as jnp
from jax import lax
from jax.experimental import pallas as pl
from jax.experimental.pallas import tpu as pltpu
from jax.experimental.pallas import tpu_sc as plsc

BATCH = 16384
DIM = 32
V = 1000000

NC, NS = 2, 16
NW = NC * NS
NWIN = (V + 127) // 128
WPT = (NWIN + NW - 1) // NW
NVREG = BATCH // 16
RING = 6
SROWS = 64
GCAP = BATCH + 16 * WPT


def _dyn_read(ref, k):
  base = (k // 16) * 16
  v = ref[pl.ds(base, 16)]
  iota = lax.iota(jnp.int32, 16)
  return jnp.sum(jnp.where(iota == (k - base), v, 0))


def _sc_body(user_ref, item_ref, gu_t, gi_t, mu_t, mi_t, uo, io,
             idxv, selp, dwin, dstart, wbuf, stage, stpos, ktmp,
             semf, sem2):
  c = lax.axis_index("c")
  s = lax.axis_index("s")
  wid = s * NC + c
  lo = wid * WPT
  hi = lo + WPT
  iota = lax.iota(jnp.int32, 16)
  lane0 = iota == 0

  for tab_a, tab_b, src_ref, out_ref in (
      (gu_t, mu_t, user_ref, uo),
      (gi_t, mi_t, item_ref, io),
  ):
    pltpu.sync_copy(src_ref, idxv)
    for par in (0, 1):
      for q in range(SROWS // 16):
        stpos[par, pl.ds(q * 16, 16)] = jnp.full((16,), -1, jnp.int32)

    def select(v, cnt):
      u = idxv[pl.ds(v * 16, 16)]
      w = lax.shift_right_logical(u, 7)
      m = (w >= lo) & (w < hi)
      plsc.store_compressed(selp.at[pl.ds(cnt, 16)], iota + v * 16, mask=m)
      return cnt + plsc.all_reduce_population_count(m)[0]

    cnt = lax.fori_loop(0, NVREG, select, 0)
    nchunks = (cnt + 511) // 512
    mtotal = nchunks * 512

    BIG = jnp.int32(0x7FFFFFF)

    def sort_chunk(ch, carry):
      dn0, lastk = carry
      cbase = ch * 512
      K = []
      Vp = []
      for v in range(32):
        posv = selp[pl.ds(cbase + v * 16, 16)] & (BATCH - 1)
        valid = (iota + cbase + v * 16) < cnt
        rv = plsc.load_gather(idxv, [posv])
        K.append(jnp.where(valid, lax.shift_right_logical(rv, 7), BIG))
        Vp.append(jnp.where(valid, posv, -1))

      def srt(v, desc):
        K[v], Vp[v] = plsc.sort_key_val(K[v], Vp[v], descending=desc)

      for v in range(32):
        srt(v, bool(v & 1))
      for kb in (2, 4, 8, 16, 32):
        jv = kb >> 1
        while jv:
          for v in range(32):
            if v & jv:
              continue
            p = v | jv
            m = K[v] <= K[p]
            if v & kb:
              m = ~m
            kl = jnp.where(m, K[v], K[p])
            kh = jnp.where(m, K[p], K[v])
            vl = jnp.where(m, Vp[v], Vp[p])
            vh = jnp.where(m, Vp[p], Vp[v])
            K[v], K[p], Vp[v], Vp[p] = kl, kh, vl, vh
          jv >>= 1
        for v in range(32):
          srt(v, bool(v & kb))

      dn = dn0
      for v in range(32):
        ktmp[...] = K[v]
        prev = jnp.where(
            iota == 0, lastk,
            plsc.load_gather(ktmp, [jnp.maximum(iota - 1, 0)]))
        m_new = (K[v] != prev) & (K[v] != BIG)
        plsc.store_compressed(dwin.at[pl.ds(dn, 16)], K[v], mask=m_new)
        plsc.store_compressed(dstart.at[pl.ds(dn, 16)],
                              iota + cbase + v * 16, mask=m_new)
        dn = dn + plsc.all_reduce_population_count(m_new)[0]
        selp[pl.ds(cbase + v * 16, 16)] = Vp[v]
        lastk = jnp.sum(jnp.where(iota == 15, K[v], 0))
      return dn, lastk

    dn, _ = lax.fori_loop(0, nchunks, sort_chunk, (0, jnp.int32(-1)))

    @pl.when(dn > 0)
    def _():
      def issue(k, b):
        win = _dyn_read(dwin, jnp.minimum(k, dn - 1))
        start = pl.multiple_of(win * 128, 128)
        pltpu.async_copy(tab_a.at[:, pl.ds(start, 128)], wbuf.at[2 * b],
                         semf[2 * b])
        pltpu.async_copy(tab_b.at[:, pl.ds(start, 128)], wbuf.at[2 * b + 1],
                         semf[2 * b + 1])

      def wait_slot(b):
        pltpu.make_async_copy(tab_a.at[:, pl.ds(0, 128)], wbuf.at[2 * b],
                              semf[2 * b]).wait()
        pltpu.make_async_copy(tab_b.at[:, pl.ds(0, 128)], wbuf.at[2 * b + 1],
                              semf[2 * b + 1]).wait()

      def flush(fc):
        for par in (0, 1):
          other = 1 - par

          @pl.when(fc % 2 == par)
          def _(par=par, other=other):
            pltpu.async_copy(
                stage.at[par],
                out_ref.at[plsc.Indices(stpos.at[par], ignored_value=-1)],
                sem2[par])

            @pl.when(fc >= 1)
            def _():
              pltpu.make_async_copy(
                  stage.at[other],
                  out_ref.at[plsc.Indices(stpos.at[other], ignored_value=-1)],
                  sem2[other]).wait()

            for q in range(SROWS // 16):
              stpos[other, pl.ds(q * 16, 16)] = jnp.full((16,), -1, jnp.int32)

      for b in range(RING):
        issue(jnp.int32(b), b)

      def extract_window(k, b, j0, fc0):
        wait_slot(b)
        live = k < dn
        kc = jnp.minimum(k, dn - 1)
        s0 = _dyn_read(dstart, kc)
        s1 = jnp.where(k + 1 < dn,
                       _dyn_read(dstart, jnp.minimum(k + 1, dn - 1)), mtotal)
        s1 = jnp.where(live, s1, s0)
        g0 = s0 // 16
        trip = (s1 + 15) // 16 - g0

        def memb16(g, carry):
          j, fc = carry
          par = fc % 2
          absb = (g0 + g) * 16
          posr = selp[pl.ds(absb, 16)]
          posc = posr & (BATCH - 1)
          rv = plsc.load_gather(idxv, [posc])
          parv = jnp.full((16,), par, jnp.int32)
          for si in range(16):
            pos_s = jnp.sum(jnp.where(iota == si, posr, 0))
            r_s = jnp.sum(jnp.where(iota == si, rv, 0))
            valid = ((absb + si) >= s0) & ((absb + si) < s1)
            pos_eff = jnp.where(valid, pos_s, -1)
            col = jnp.full((16,), r_s & 127, jnp.int32)
            row = jnp.full((16,), j + si, jnp.int32)
            for h in range(2):
              ridx = iota + 16 * h
              va = plsc.load_gather(wbuf.at[2 * b], [ridx, col])
              vb = plsc.load_gather(wbuf.at[2 * b + 1], [ridx, col])
              plsc.store_scatter(stage, [parv, row, ridx], va)
              plsc.store_scatter(stage, [parv, row, ridx + DIM], vb)
            plsc.store_scatter(stpos, [parv, row],
                               jnp.full((16,), pos_eff, jnp.int32),
                               mask=lane0)
          j = j + 16

          @pl.when(j == SROWS)
          def _():
            flush(fc)

          wrap = j == SROWS
          return jnp.where(wrap, 0, j), fc + jnp.where(wrap, 1, 0)

        j, fc = lax.fori_loop(0, trip, memb16, (j0, fc0))
        issue(k + RING, b)
        return j, fc

      def ring_step(k0, carry):
        j, fc = carry
        for b in range(RING):
          j, fc = extract_window(k0 * RING + b, b, j, fc)
        return j, fc

      nsteps = (dn + RING - 1) // RING
      j, fc = lax.fori_loop(0, nsteps, ring_step, (0, 0))

      for b in range(RING):
        wait_slot(b)

      @pl.when(j > 0)
      def _():
        flush(fc)

      fct = fc + jnp.where(j > 0, 1, 0)
      for par in (0, 1):
        @pl.when((fct >= 1) & ((fct - 1) % 2 == par))
        def _(par=par):
          pltpu.make_async_copy(
              stage.at[par],
              out_ref.at[plsc.Indices(stpos.at[par], ignored_value=-1)],
              sem2[par]).wait()


def _sc_gather(user, item, gu_t, gi_t, mu_t, mi_t):
  mesh = plsc.VectorSubcoreMesh(core_axis_name="c", subcore_axis_name="s",
                                num_cores=NC, num_subcores=NS)
  f = pl.kernel(
      _sc_body,
      out_type=[
          jax.ShapeDtypeStruct((BATCH, 128), jnp.float32),
          jax.ShapeDtypeStruct((BATCH, 128), jnp.float32),
      ],
      mesh=mesh,
      scratch_types=[
          pltpu.VMEM((BATCH,), jnp.int32),
          pltpu.VMEM((BATCH,), jnp.int32),
          pltpu.VMEM((8192,), jnp.int32),
          pltpu.VMEM((8192,), jnp.int32),
          pltpu.VMEM((RING * 2, DIM, 128), jnp.float32),
          pltpu.VMEM((2, SROWS, 128), jnp.float32),
          pltpu.VMEM((2, SROWS), jnp.int32),
          pltpu.VMEM((16,), jnp.int32),
          [pltpu.SemaphoreType.DMA] * (RING * 2),
          [pltpu.SemaphoreType.DMA] * 2,
      ],
      compiler_params=pltpu.CompilerParams(needs_layout_passes=False),
  )
  return f(user, item, gu_t, gi_t, mu_t, mi_t)


def _tc_body(u_ref, i_ref, w1_ref, b1_ref, w2_ref, b2_ref,
             w3_ref, b3_ref, wo_ref, bo_ref, out_ref):
  u = u_ref[...]
  it = i_ref[...]
  guv = u[:, :DIM] * it[:, :DIM]
  h = jnp.concatenate([u[:, DIM:2 * DIM], it[:, DIM:2 * DIM]], axis=1)
  dn = (((1,), (1,)), ((), ()))
  h = jnp.maximum(
      lax.dot_general(h, w1_ref[...], dn,
                      preferred_element_type=jnp.float32) + b1_ref[...], 0.0)
  h = jnp.maximum(
      lax.dot_general(h, w2_ref[...], dn,
                      preferred_element_type=jnp.float32) + b2_ref[...], 0.0)
  h = jnp.maximum(
      lax.dot_general(h, w3_ref[...], dn,
                      preferred_element_type=jnp.float32) + b3_ref[...], 0.0)
  wo = wo_ref[...]
  dot = lax.dot_general(guv, wo[:, :DIM], dn,
                        preferred_element_type=jnp.float32)
  dot = dot + lax.dot_general(h, wo[:, DIM:], dn,
                              preferred_element_type=jnp.float32)
  out_ref[...] = dot + bo_ref[0, 0]


def _tc_mlp(u, i, w1, b1, w2, b2, w3, b3, wo, bo):
  nblk = 8
  blk = BATCH // nblk
  data_spec = pl.BlockSpec((blk, 128), lambda k: (k, 0))
  full = lambda shape: pl.BlockSpec(shape, lambda k: (0, 0))
  return pl.pallas_call(
      _tc_body,
      grid=(nblk,),
      in_specs=[
          data_spec, data_spec,
          full(w1.shape), full(b1.shape),
          full(w2.shape), full(b2.shape),
          full(w3.shape), full(b3.shape),
          full(wo.shape), full(bo.shape),
      ],
      out_specs=pl.BlockSpec((blk, 1), lambda k: (k, 0)),
      out_shape=jax.ShapeDtypeStruct((BATCH, 1), jnp.float32),
  )(u, i, w1, b1, w2, b2, w3, b3, wo, bo)


@jax.jit
def kernel(user, item, GMF_U, GMF_I, MLP_U, MLP_I,
           W1, b1, W2, b2, W3, b3, Wo, bo):
  u, i = _sc_gather(user, item, GMF_U.T, GMF_I.T, MLP_U.T, MLP_I.T)
  out = _tc_mlp(u, i,
                W1, b1.reshape(1, -1), W2, b2.reshape(1, -1),
                W3, b3.reshape(1, -1), Wo, bo.reshape(1, 1))
  return out.reshape(-1)


# speedup vs baseline: 1.3381x; 1.3381x over previous
import jax
import jax.numpy as jnp
from jax import lax
from jax.experimental import pallas as pl
from jax.experimental.pallas import tpu as pltpu
from jax.experimental.pallas import tpu_sc as plsc

BATCH = 16384
DIM = 32

NC, NS = 2, 16
NW = NC * NS
CHUNK = BATCH // NW
NGRP = CHUNK // 16
NBUF = 4


def _sc_body(user_ref, item_ref, gu_t, gi_t, mu_t, mi_t,
             muo, mio, guvo,
             ivu, ivi, wb, stmu, stmi, stguv, semf):
  c = lax.axis_index("c")
  s = lax.axis_index("s")
  wid = s * NC + c
  base = wid * CHUNK
  pltpu.sync_copy(user_ref.at[pl.ds(base, CHUNK)], ivu)
  pltpu.sync_copy(item_ref.at[pl.ds(base, CHUNK)], ivi)
  iota = lax.iota(jnp.int32, 16)

  tabs = (gu_t, gi_t, mu_t, mi_t)

  def issue(si, uvec, ivec):
    ru = uvec[si]
    ri = ivec[si]
    slot = si % NBUF
    for t in range(4):
      r = ru if t in (0, 2) else ri
      start = pl.multiple_of((r // 128) * 128, 128)
      pltpu.async_copy(tabs[t].at[:, pl.ds(start, 128)],
                       wb.at[slot * 4 + t], semf[slot])

  def wait_slot(slot):
    for t in range(4):
      pltpu.make_async_copy(tabs[t].at[:, pl.ds(0, 128)],
                            wb.at[slot * 4 + t], semf[slot]).wait()

  def extract(lane, uvec, ivec, pos_scalar, slot):
    cu = jnp.full((16,), uvec[lane] & 127, jnp.int32)
    ci = jnp.full((16,), ivec[lane] & 127, jnp.int32)
    pos = jnp.full((16,), pos_scalar, jnp.int32)
    for h in range(2):
      ridx = iota + 16 * h
      vgu = plsc.load_gather(wb.at[slot * 4 + 0], [ridx, cu])
      vgi = plsc.load_gather(wb.at[slot * 4 + 1], [ridx, ci])
      vmu = plsc.load_gather(wb.at[slot * 4 + 2], [ridx, cu])
      vmi = plsc.load_gather(wb.at[slot * 4 + 3], [ridx, ci])
      plsc.store_scatter(stguv, [ridx, pos], vgu * vgi)
      plsc.store_scatter(stmu, [ridx, pos], vmu)
      plsc.store_scatter(stmi, [ridx, pos], vmi)

  def group(g, carry):
    uvec = ivu[pl.ds(g * 16, 16)]
    ivec = ivi[pl.ds(g * 16, 16)]
    pb = jnp.maximum(g * 16 - 16, 0)
    puvec = ivu[pl.ds(pb, 16)]
    pivec = ivi[pl.ds(pb, 16)]
    for si in range(16):
      slot = si % NBUF
      if si < NBUF:
        @pl.when(g > 0)
        def _(si=si, slot=slot):
          wait_slot(slot)
          extract(si + 16 - NBUF, puvec, pivec, g * 16 + si - NBUF, slot)
      else:
        wait_slot(slot)
        extract(si - NBUF, uvec, ivec, g * 16 + si - NBUF, slot)
      issue(si, uvec, ivec)
    return carry

  lax.fori_loop(0, NGRP, group, 0)

  lvec_u = ivu[pl.ds(CHUNK - 16, 16)]
  lvec_i = ivi[pl.ds(CHUNK - 16, 16)]
  for si in range(NBUF):
    wait_slot(si % NBUF)
    extract(si + 16 - NBUF, lvec_u, lvec_i, CHUNK - NBUF + si, si % NBUF)

  lane = pl.ds(base, CHUNK)
  pltpu.sync_copy(stmu, muo.at[:, lane])
  pltpu.sync_copy(stmi, mio.at[:, lane])
  pltpu.sync_copy(stguv, guvo.at[:, lane])


def _sc_gather(user, item, gu_t, gi_t, mu_t, mi_t):
  mesh = plsc.VectorSubcoreMesh(core_axis_name="c", subcore_axis_name="s",
                                num_cores=NC, num_subcores=NS)
  f = pl.kernel(
      _sc_body,
      out_type=[
          jax.ShapeDtypeStruct((DIM, BATCH), jnp.float32),
          jax.ShapeDtypeStruct((DIM, BATCH), jnp.float32),
          jax.ShapeDtypeStruct((DIM, BATCH), jnp.float32),
      ],
      mesh=mesh,
      scratch_types=[
          pltpu.VMEM((CHUNK,), jnp.int32),
          pltpu.VMEM((CHUNK,), jnp.int32),
          pltpu.VMEM((4 * NBUF, DIM, 128), jnp.float32),
          pltpu.VMEM((DIM, CHUNK), jnp.float32),
          pltpu.VMEM((DIM, CHUNK), jnp.float32),
          pltpu.VMEM((DIM, CHUNK), jnp.float32),
          [pltpu.SemaphoreType.DMA] * NBUF,
      ],
      compiler_params=pltpu.CompilerParams(needs_layout_passes=False),
  )
  return f(user, item, gu_t, gi_t, mu_t, mi_t)


def _tc_body(mu_ref, mi_ref, guv_ref, w1_ref, b1_ref, w2_ref, b2_ref,
             w3_ref, b3_ref, wo_ref, bo_ref, out_ref):
  h = jnp.concatenate([mu_ref[...], mi_ref[...]], axis=0)
  dn = (((1,), (0,)), ((), ()))
  h = jnp.maximum(
      lax.dot_general(w1_ref[...], h, dn,
                      preferred_element_type=jnp.float32) + b1_ref[...], 0.0)
  h = jnp.maximum(
      lax.dot_general(w2_ref[...], h, dn,
                      preferred_element_type=jnp.float32) + b2_ref[...], 0.0)
  h = jnp.maximum(
      lax.dot_general(w3_ref[...], h, dn,
                      preferred_element_type=jnp.float32) + b3_ref[...], 0.0)
  wo = wo_ref[...]
  dot = lax.dot_general(wo[:, :DIM], guv_ref[...], dn,
                        preferred_element_type=jnp.float32)
  dot = dot + lax.dot_general(wo[:, DIM:], h, dn,
                              preferred_element_type=jnp.float32)
  out_ref[...] = dot + bo_ref[0, 0]


def _tc_mlp(mu, mi, guv, w1, b1, w2, b2, w3, b3, wo, bo):
  nblk = 8
  blk = BATCH // nblk
  data_spec = pl.BlockSpec((DIM, blk), lambda i: (0, i))
  full = lambda shape: pl.BlockSpec(shape, lambda i: (0, 0))
  return pl.pallas_call(
      _tc_body,
      grid=(nblk,),
      in_specs=[
          data_spec, data_spec, data_spec,
          full(w1.shape), full(b1.shape),
          full(w2.shape), full(b2.shape),
          full(w3.shape), full(b3.shape),
          full(wo.shape), full(bo.shape),
      ],
      out_specs=pl.BlockSpec((1, blk), lambda i: (0, i)),
      out_shape=jax.ShapeDtypeStruct((1, BATCH), jnp.float32),
  )(mu, mi, guv, w1, b1, w2, b2, w3, b3, wo, bo)


@jax.jit
def kernel(user, item, GMF_U, GMF_I, MLP_U, MLP_I,
           W1, b1, W2, b2, W3, b3, Wo, bo):
  mu, mi, guv = _sc_gather(user, item, GMF_U.T, GMF_I.T, MLP_U.T, MLP_I.T)
  out = _tc_mlp(mu, mi, guv,
                W1, b1.reshape(-1, 1), W2, b2.reshape(-1, 1),
                W3, b3.reshape(-1, 1), Wo, bo.reshape(1, 1))
  return out.reshape(-1)


# speedup vs baseline: 1.4374x; 1.0742x over previous
import jax
import jax.numpy as jnp
from jax import lax
from jax.experimental import pallas as pl
from jax.experimental.pallas import tpu as pltpu
from jax.experimental.pallas import tpu_sc as plsc

BATCH = 16384
DIM = 32

NC, NS = 2, 16
NW = NC * NS
CHUNK = BATCH // NW
NGRP = CHUNK // 16
NBUF = 4


def _sc_body(user_ref, item_ref, gu_t, gi_t, mu_t, mi_t,
             muo, mio, guvo,
             ivu, ivi, fidx, wb, stmu, stmi, stguv, sem):
  c = lax.axis_index("c")
  s = lax.axis_index("s")
  wid = s * NC + c
  base = wid * CHUNK
  pltpu.sync_copy(user_ref.at[pl.ds(base, CHUNK)], ivu)
  pltpu.sync_copy(item_ref.at[pl.ds(base, CHUNK)], ivi)
  iota = lax.iota(jnp.int32, 16)
  fidx[pl.ds(0, 16)] = iota
  fidx[pl.ds(16, 16)] = iota + 16

  tabs = (gu_t, gi_t, mu_t, mi_t)

  def group(g, carry):
    uvec = ivu[pl.ds(g * 16, 16)]
    ivec = ivi[pl.ds(g * 16, 16)]

    def issue(si):
      ru = uvec[si]
      ri = ivec[si]
      slot = si % NBUF
      cps = []
      cols = []
      for t in range(4):
        r = ru if t in (0, 2) else ri
        start = pl.multiple_of((r // 128) * 128, 128)
        cols.append(r - start)
        cps.append(pltpu.async_copy(
            tabs[t].at[:, pl.ds(start, 128)], wb.at[slot * 4 + t], sem))
      return cps, cols

    def extract(si, cols):
      slot = si % NBUF
      pos = jnp.full((16,), g * 16 + si, jnp.int32)
      for h in range(2):
        ridx = iota + 16 * h
        cu = jnp.full((16,), cols[0], jnp.int32)
        ci = jnp.full((16,), cols[1], jnp.int32)
        vgu = plsc.load_gather(wb.at[slot * 4 + 0], [ridx, cu])
        vgi = plsc.load_gather(wb.at[slot * 4 + 1], [ridx, ci])
        vmu = plsc.load_gather(wb.at[slot * 4 + 2], [ridx, cu])
        vmi = plsc.load_gather(wb.at[slot * 4 + 3], [ridx, ci])
        plsc.store_scatter(stguv, [ridx, pos], vgu * vgi)
        plsc.store_scatter(stmu, [ridx, pos], vmu)
        plsc.store_scatter(stmi, [ridx, pos], vmi)

    pend = [None] * NBUF
    for si in range(16):
      if pend[si % NBUF] is not None:
        pcps, pcols, psi = pend[si % NBUF]
        for cp in pcps:
          cp.wait()
        extract(psi, pcols)
      cps, cols = issue(si)
      pend[si % NBUF] = (cps, cols, si)
    for k in range(NBUF):
      pcps, pcols, psi = pend[(16 + k) % NBUF]
      for cp in pcps:
        cp.wait()
      extract(psi, pcols)
    return carry

  lax.fori_loop(0, NGRP, group, 0)

  lane = pl.ds(base, CHUNK)
  pltpu.sync_copy(stmu, muo.at[:, lane])
  pltpu.sync_copy(stmi, mio.at[:, lane])
  pltpu.sync_copy(stguv, guvo.at[:, lane])


def _sc_gather(user, item, gu_t, gi_t, mu_t, mi_t):
  mesh = plsc.VectorSubcoreMesh(core_axis_name="c", subcore_axis_name="s",
                                num_cores=NC, num_subcores=NS)
  f = pl.kernel(
      _sc_body,
      out_type=[
          jax.ShapeDtypeStruct((DIM, BATCH), jnp.float32),
          jax.ShapeDtypeStruct((DIM, BATCH), jnp.float32),
          jax.ShapeDtypeStruct((DIM, BATCH), jnp.float32),
      ],
      mesh=mesh,
      scratch_types=[
          pltpu.VMEM((CHUNK,), jnp.int32),
          pltpu.VMEM((CHUNK,), jnp.int32),
          pltpu.VMEM((DIM,), jnp.int32),
          pltpu.VMEM((4 * NBUF, DIM, 128), jnp.float32),
          pltpu.VMEM((DIM, CHUNK), jnp.float32),
          pltpu.VMEM((DIM, CHUNK), jnp.float32),
          pltpu.VMEM((DIM, CHUNK), jnp.float32),
          pltpu.SemaphoreType.DMA,
      ],
      compiler_params=pltpu.CompilerParams(needs_layout_passes=False),
  )
  return f(user, item, gu_t, gi_t, mu_t, mi_t)


def _tc_body(mu_ref, mi_ref, guv_ref, w1_ref, b1_ref, w2_ref, b2_ref,
             w3_ref, b3_ref, wo_ref, bo_ref, out_ref):
  h = jnp.concatenate([mu_ref[...], mi_ref[...]], axis=0)
  dn = (((1,), (0,)), ((), ()))
  h = jnp.maximum(
      lax.dot_general(w1_ref[...], h, dn,
                      preferred_element_type=jnp.float32) + b1_ref[...], 0.0)
  h = jnp.maximum(
      lax.dot_general(w2_ref[...], h, dn,
                      preferred_element_type=jnp.float32) + b2_ref[...], 0.0)
  h = jnp.maximum(
      lax.dot_general(w3_ref[...], h, dn,
                      preferred_element_type=jnp.float32) + b3_ref[...], 0.0)
  wo = wo_ref[...]
  dot = lax.dot_general(wo[:, :DIM], guv_ref[...], dn,
                        preferred_element_type=jnp.float32)
  dot = dot + lax.dot_general(wo[:, DIM:], h, dn,
                              preferred_element_type=jnp.float32)
  out_ref[...] = dot + bo_ref[0, 0]


def _tc_mlp(mu, mi, guv, w1, b1, w2, b2, w3, b3, wo, bo):
  nblk = 8
  blk = BATCH // nblk
  data_spec = pl.BlockSpec((DIM, blk), lambda i: (0, i))
  full = lambda shape: pl.BlockSpec(shape, lambda i: (0, 0))
  return pl.pallas_call(
      _tc_body,
      grid=(nblk,),
      in_specs=[
          data_spec, data_spec, data_spec,
          full(w1.shape), full(b1.shape),
          full(w2.shape), full(b2.shape),
          full(w3.shape), full(b3.shape),
          full(wo.shape), full(bo.shape),
      ],
      out_specs=pl.BlockSpec((1, blk), lambda i: (0, i)),
      out_shape=jax.ShapeDtypeStruct((1, BATCH), jnp.float32),
  )(mu, mi, guv, w1, b1, w2, b2, w3, b3, wo, bo)


@jax.jit
def kernel(user, item, GMF_U, GMF_I, MLP_U, MLP_I,
           W1, b1, W2, b2, W3, b3, Wo, bo):
  mu, mi, guv = _sc_gather(user, item, GMF_U.T, GMF_I.T, MLP_U.T, MLP_I.T)
  out = _tc_mlp(mu, mi, guv,
                W1, b1.reshape(-1, 1), W2, b2.reshape(-1, 1),
                W3, b3.reshape(-1, 1), Wo, bo.reshape(1, 1))
  return out.reshape(-1)
